# Initial kernel scaffold; baseline (speedup 1.0000x reference)
import jax
import jax.numpy as jnp
from jax.experimental import pallas as pl


def kernel(features, edge_index, W1, b1, W2, b2):
    raise NotImplementedError("write your pallas kernel here")


# speedup vs baseline: 14.6571x; 14.6571x over previous
import functools

import jax
import jax.numpy as jnp
from jax import lax
from jax.experimental import pallas as pl
from jax.experimental.pallas import tpu as pltpu
from jax.experimental.pallas import tpu_sc as plsc

N_NODES = 10000
D_IN = 128
D_HID = 200
N_CLS = 40
D2 = 48
N_PAD = 10240
N_TILES = 32
ROWS_PER_TILE = N_PAD // 16
CH = 128

_MESH = plsc.VectorSubcoreMesh(core_axis_name="c", subcore_axis_name="s")
_SC_PARAMS = pltpu.CompilerParams(use_tc_tiling_on_sc=False)


def _sc_degree(dst3, ones, zeros, nch):
    @functools.partial(
        pl.kernel,
        out_type=jax.ShapeDtypeStruct((2, N_PAD, 16), jnp.float32),
        mesh=_MESH,
        compiler_params=_SC_PARAMS,
        scratch_types=[
            pltpu.VMEM((nch, CH), jnp.int32),
            pltpu.VMEM((CH, 16), jnp.float32),
            pltpu.VMEM_SHARED((N_PAD, 16), jnp.float32),
        ],
    )
    def k(dst_hbm, ones_hbm, z_hbm, out_hbm, dst_v, ones_v, acc):
        c = lax.axis_index("c")
        s = lax.axis_index("s")
        wid = s * 2 + c
        row0 = s * ROWS_PER_TILE
        pltpu.sync_copy(dst_hbm.at[wid], dst_v)
        pltpu.sync_copy(ones_hbm, ones_v)
        pltpu.sync_copy(z_hbm, acc.at[pl.ds(row0, ROWS_PER_TILE)])
        plsc.subcore_barrier()

        @pl.loop(0, nch)
        def _(j):
            pltpu.sync_copy(ones_v, acc.at[dst_v.at[j]], add=True)

        plsc.subcore_barrier()
        pltpu.sync_copy(
            acc.at[pl.ds(row0, ROWS_PER_TILE)],
            out_hbm.at[c, pl.ds(row0, ROWS_PER_TILE)],
        )

    return k(dst3, ones, zeros)


def _sc_segment_sum(g, src3, dst3, zeros, d, nch):
    @functools.partial(
        pl.kernel,
        out_type=jax.ShapeDtypeStruct((2, N_PAD, d), jnp.float32),
        mesh=_MESH,
        compiler_params=_SC_PARAMS,
        scratch_types=[
            pltpu.VMEM((nch, CH), jnp.int32),
            pltpu.VMEM((nch, CH), jnp.int32),
            pltpu.VMEM((CH, d), jnp.float32),
            pltpu.VMEM_SHARED((N_PAD, d), jnp.float32),
        ],
    )
    def k(g_hbm, src_hbm, dst_hbm, z_hbm, out_hbm, src_v, dst_v, gbuf, acc):
        c = lax.axis_index("c")
        s = lax.axis_index("s")
        wid = s * 2 + c
        row0 = s * ROWS_PER_TILE
        pltpu.sync_copy(src_hbm.at[wid], src_v)
        pltpu.sync_copy(dst_hbm.at[wid], dst_v)
        pltpu.sync_copy(z_hbm, acc.at[pl.ds(row0, ROWS_PER_TILE)])
        plsc.subcore_barrier()

        @pl.loop(0, nch)
        def _(j):
            pltpu.sync_copy(g_hbm.at[src_v.at[j]], gbuf)
            pltpu.sync_copy(gbuf, acc.at[dst_v.at[j]], add=True)

        plsc.subcore_barrier()
        pltpu.sync_copy(
            acc.at[pl.ds(row0, ROWS_PER_TILE)],
            out_hbm.at[c, pl.ds(row0, ROWS_PER_TILE)],
        )

    return k(g, src3, dst3, zeros)


_R = 512


def _tc_prescale(degp, feat):
    def body(degp_ref, feat_ref, d_ref, g1_ref):
        dp = degp_ref[...]
        deg = dp[0, :, 0:1] + dp[1, :, 0:1] + 1.0
        dcol = lax.rsqrt(deg)
        dbc = jnp.broadcast_to(dcol, (_R, 128))
        d_ref[...] = dbc
        g1_ref[...] = feat_ref[...] * dbc

    return pl.pallas_call(
        body,
        grid=(N_PAD // _R,),
        in_specs=[
            pl.BlockSpec((2, _R, 16), lambda i: (0, i, 0)),
            pl.BlockSpec((_R, 128), lambda i: (i, 0)),
        ],
        out_specs=[
            pl.BlockSpec((_R, 128), lambda i: (i, 0)),
            pl.BlockSpec((_R, 128), lambda i: (i, 0)),
        ],
        out_shape=[
            jax.ShapeDtypeStruct((N_PAD, 128), jnp.float32),
            jax.ShapeDtypeStruct((N_PAD, 128), jnp.float32),
        ],
    )(degp, feat)


def _tc_layer(agg1p, g1, d, W1, b1, W2p):
    def body(ap_ref, g1_ref, d_ref, w1_ref, b1_ref, w2_ref, g2_ref):
        ap = ap_ref[...]
        dbc = d_ref[...]
        t = (ap[0] + ap[1] + g1_ref[...]) * dbc
        x1 = jnp.dot(t, w1_ref[...], preferred_element_type=jnp.float32)
        x1 = jnp.maximum(x1 + b1_ref[...], 0.0)
        h2 = jnp.dot(x1, w2_ref[...], preferred_element_type=jnp.float32)
        g2_ref[...] = h2 * dbc[:, :D2]

    return pl.pallas_call(
        body,
        grid=(N_PAD // _R,),
        in_specs=[
            pl.BlockSpec((2, _R, 128), lambda i: (0, i, 0)),
            pl.BlockSpec((_R, 128), lambda i: (i, 0)),
            pl.BlockSpec((_R, 128), lambda i: (i, 0)),
            pl.BlockSpec((D_IN, D_HID), lambda i: (0, 0)),
            pl.BlockSpec((1, D_HID), lambda i: (0, 0)),
            pl.BlockSpec((D_HID, D2), lambda i: (0, 0)),
        ],
        out_specs=pl.BlockSpec((_R, D2), lambda i: (i, 0)),
        out_shape=jax.ShapeDtypeStruct((N_PAD, D2), jnp.float32),
    )(agg1p, g1, d, W1, b1, W2p)


def _tc_final(agg2p, g2, d, b2):
    def body(ap_ref, g2_ref, d_ref, b2_ref, o_ref):
        ap = ap_ref[...]
        t = (ap[0] + ap[1] + g2_ref[...]) * d_ref[:, :D2]
        o_ref[...] = jnp.maximum(t + b2_ref[...], 0.0)

    return pl.pallas_call(
        body,
        grid=(N_PAD // _R,),
        in_specs=[
            pl.BlockSpec((2, _R, D2), lambda i: (0, i, 0)),
            pl.BlockSpec((_R, D2), lambda i: (i, 0)),
            pl.BlockSpec((_R, 128), lambda i: (i, 0)),
            pl.BlockSpec((1, D2), lambda i: (0, 0)),
        ],
        out_specs=pl.BlockSpec((_R, D2), lambda i: (i, 0)),
        out_shape=jax.ShapeDtypeStruct((N_PAD, D2), jnp.float32),
    )(agg2p, g2, d, b2)


def kernel(features, edge_index, W1, b1, W2, b2):
    e = edge_index.shape[1]
    nch = -(-e // (N_TILES * CH))
    e_pad = N_TILES * nch * CH - e

    src = edge_index[0].astype(jnp.int32)
    dst = edge_index[1].astype(jnp.int32)
    src3 = jnp.pad(src, (0, e_pad)).reshape(N_TILES, nch, CH)
    dst3 = jnp.pad(dst, (0, e_pad), constant_values=N_NODES).reshape(N_TILES, nch, CH)

    feat_pad = jnp.pad(features, ((0, N_PAD - N_NODES), (0, 0)))
    W2p = jnp.pad(W2, ((0, 0), (0, D2 - N_CLS)))
    b1r = b1.reshape(1, D_HID)
    b2r = jnp.pad(b2, (0, D2 - N_CLS)).reshape(1, D2)
    ones16 = jnp.ones((CH, 16), jnp.float32)
    zeros16 = jnp.zeros((ROWS_PER_TILE, 16), jnp.float32)
    zeros128 = jnp.zeros((ROWS_PER_TILE, 128), jnp.float32)
    zeros48 = jnp.zeros((ROWS_PER_TILE, D2), jnp.float32)

    degp = _sc_degree(dst3, ones16, zeros16, nch)
    d_bcast, g1 = _tc_prescale(degp, feat_pad)
    agg1p = _sc_segment_sum(g1, src3, dst3, zeros128, 128, nch)
    g2 = _tc_layer(agg1p, g1, d_bcast, W1, b1r, W2p)
    agg2p = _sc_segment_sum(g2, src3, dst3, zeros48, D2, nch)
    out = _tc_final(agg2p, g2, d_bcast, b2r)
    return out[:N_NODES, :N_CLS]
